# baseline (speedup 1.0000x reference)
import functools

import jax
import jax.numpy as jnp
from jax import lax
from jax.experimental import pallas as pl
from jax.experimental.pallas import tpu as pltpu
from jax.experimental.pallas import tpu_sc as plsc

B = 4096
D = 64
NUM_CORES = 2
NUM_SUBCORES = 16
LANES = 16
NW = NUM_CORES * NUM_SUBCORES
BPW = B // NW
GROUPS = BPW // LANES

_mesh = plsc.VectorSubcoreMesh(core_axis_name="c", subcore_axis_name="s")


@functools.partial(
    pl.kernel,
    out_type=jax.ShapeDtypeStruct((B,), jnp.float32),
    mesh=_mesh,
    compiler_params=pltpu.CompilerParams(needs_layout_passes=False),
    scratch_types=[
        pltpu.VMEM((BPW,), jnp.int32),
        pltpu.VMEM((BPW,), jnp.int32),
        pltpu.VMEM((BPW,), jnp.int32),
        pltpu.VMEM((BPW,), jnp.int32),
        pltpu.VMEM((BPW,), jnp.int32),
        pltpu.VMEM((BPW,), jnp.int32),
        pltpu.VMEM((BPW, 2 * D), jnp.float32),
        pltpu.VMEM((BPW, 2 * D), jnp.float32),
        pltpu.VMEM((BPW, 2 * D), jnp.float32),
        pltpu.VMEM((BPW,), jnp.float32),
        pltpu.SemaphoreType.DMA,
        pltpu.SemaphoreType.DMA,
        pltpu.SemaphoreType.DMA,
    ],
)
def _l1_score_kernel(heads, rels, tails, etab2, rtab2, out,
                     hraw, rraw, traw, hpair, rpair, tpair,
                     hrows, rrows, trows, sums,
                     sem_h, sem_r, sem_t):
    wid = lax.axis_index("s") * NUM_CORES + lax.axis_index("c")
    base = wid * BPW

    pltpu.sync_copy(heads.at[pl.ds(base, BPW)], hraw)
    pltpu.sync_copy(rels.at[pl.ds(base, BPW)], rraw)
    pltpu.sync_copy(tails.at[pl.ds(base, BPW)], traw)

    for g in range(GROUPS):
        sl = pl.ds(g * LANES, LANES)
        hpair[sl] = lax.shift_right_logical(hraw[sl], 1)
        rpair[sl] = lax.shift_right_logical(rraw[sl], 1)
        tpair[sl] = lax.shift_right_logical(traw[sl], 1)

    ch = pltpu.async_copy(etab2.at[hpair], hrows, sem_h)
    cr = pltpu.async_copy(rtab2.at[rpair], rrows, sem_r)
    ct = pltpu.async_copy(etab2.at[tpair], trows, sem_t)
    ch.wait()
    cr.wait()
    ct.wait()

    lanes = lax.iota(jnp.int32, LANES)
    one = jnp.full((LANES,), 1, dtype=jnp.int32)
    for g in range(GROUPS):
        sl = pl.ds(g * LANES, LANES)
        rows = lanes + g * LANES
        hoff = lax.bitwise_and(hraw[sl], one) * D
        roff = lax.bitwise_and(rraw[sl], one) * D
        toff = lax.bitwise_and(traw[sl], one) * D

        def body(d, acc):
            dv = jnp.full((LANES,), d, dtype=jnp.int32)
            h = plsc.load_gather(hrows, [rows, hoff + dv])
            r = plsc.load_gather(rrows, [rows, roff + dv])
            t = plsc.load_gather(trows, [rows, toff + dv])
            return acc + jnp.abs(h + r - t)

        acc = lax.fori_loop(0, D, body, jnp.zeros((LANES,), jnp.float32))
        sums[sl] = acc

    pltpu.sync_copy(sums, out.at[pl.ds(base, BPW)])


def kernel(sample, entity_embedding, relation_embedding, gamma):
    heads = sample[:, 0]
    rels = sample[:, 1]
    tails = sample[:, 2]
    etab2 = entity_embedding.reshape(-1, 2 * D)
    rtab2 = relation_embedding.reshape(-1, 2 * D)
    sums = _l1_score_kernel(heads, rels, tails, etab2, rtab2)
    return (gamma - sums)[:, None]


# speedup vs baseline: 2.4983x; 2.4983x over previous
import functools

import jax
import jax.numpy as jnp
from jax import lax
from jax.experimental import pallas as pl
from jax.experimental.pallas import tpu as pltpu
from jax.experimental.pallas import tpu_sc as plsc

B = 4096
D = 64
SUB = 8
NUM_CORES = 2
NUM_SUBCORES = 16
LANES = 16
NW = NUM_CORES * NUM_SUBCORES
BPW = B // NW
GROUPS = BPW // LANES
UNROLL = 4

_mesh = plsc.VectorSubcoreMesh(core_axis_name="c", subcore_axis_name="s")


@functools.partial(
    pl.kernel,
    out_type=jax.ShapeDtypeStruct((B,), jnp.float32),
    mesh=_mesh,
    compiler_params=pltpu.CompilerParams(needs_layout_passes=False),
    scratch_types=[
        pltpu.VMEM((BPW,), jnp.int32),
        pltpu.VMEM((BPW,), jnp.int32),
        pltpu.VMEM((BPW,), jnp.int32),
        pltpu.VMEM((BPW, D), jnp.float32),
        pltpu.VMEM((BPW, D), jnp.float32),
        pltpu.VMEM((BPW, D), jnp.float32),
        pltpu.VMEM((BPW,), jnp.float32),
        pltpu.SemaphoreType.DMA,
        pltpu.SemaphoreType.DMA,
        pltpu.SemaphoreType.DMA,
    ],
)
def _l1_score_kernel(heads, rels, tails, etab, rtab, out,
                     hraw, rraw, traw,
                     hrows, rrows, trows, sums,
                     sem_h, sem_r, sem_t):
    wid = lax.axis_index("s") * NUM_CORES + lax.axis_index("c")
    base = wid * BPW

    pltpu.sync_copy(heads.at[pl.ds(base, BPW)], hraw)
    pltpu.sync_copy(rels.at[pl.ds(base, BPW)], rraw)
    pltpu.sync_copy(tails.at[pl.ds(base, BPW)], traw)

    for g in range(GROUPS):
        sl = pl.ds(g * LANES, LANES)
        hv = hraw[sl]
        rv = rraw[sl]
        tv = traw[sl]
        for j in range(LANES):
            i = g * LANES + j
            pltpu.async_copy(
                etab.at[lax.shift_right_logical(hv[j], 3),
                        lax.bitwise_and(hv[j], 7)],
                hrows.at[i], sem_h)
            pltpu.async_copy(
                rtab.at[lax.shift_right_logical(rv[j], 3),
                        lax.bitwise_and(rv[j], 7)],
                rrows.at[i], sem_r)
            pltpu.async_copy(
                etab.at[lax.shift_right_logical(tv[j], 3),
                        lax.bitwise_and(tv[j], 7)],
                trows.at[i], sem_t)

    dummy = etab.at[0, 0]

    def drain(i, _):
        pltpu.make_async_copy(dummy, hrows.at[i], sem_h).wait()
        pltpu.make_async_copy(dummy, rrows.at[i], sem_r).wait()
        pltpu.make_async_copy(dummy, trows.at[i], sem_t).wait()
        return 0

    lax.fori_loop(0, BPW, drain, 0)

    lanes = lax.iota(jnp.int32, LANES)
    for g in range(GROUPS):
        sl = pl.ds(g * LANES, LANES)
        rows = lanes + g * LANES

        def body(k, acc):
            d0 = k * UNROLL
            for u in range(UNROLL):
                col = jnp.full((LANES,), d0 + u, dtype=jnp.int32)
                h = plsc.load_gather(hrows, [rows, col])
                r = plsc.load_gather(rrows, [rows, col])
                t = plsc.load_gather(trows, [rows, col])
                acc = acc + jnp.abs(h + r - t)
            return acc

        acc = lax.fori_loop(0, D // UNROLL, body,
                            jnp.zeros((LANES,), jnp.float32))
        sums[sl] = acc

    pltpu.sync_copy(sums, out.at[pl.ds(base, BPW)])


def kernel(sample, entity_embedding, relation_embedding, gamma):
    heads = sample[:, 0]
    rels = sample[:, 1]
    tails = sample[:, 2]
    etab3 = entity_embedding.reshape(-1, SUB, D)
    rtab3 = relation_embedding.reshape(-1, SUB, D)
    sums = _l1_score_kernel(heads, rels, tails, etab3, rtab3)
    return (gamma - sums)[:, None]
